# baseline (speedup 1.0000x reference)
import functools

import jax
import jax.numpy as jnp
import numpy as np
from jax import lax
from jax.experimental import pallas as pl
from jax.experimental.pallas import tpu as pltpu
from jax.experimental.pallas import tpu_sc as plsc

VOCAB = 100000
D_MODEL = 128
MAX_LEN = 512
B = 1024
L = 200

NUM_CORES = 2
NUM_SUBCORES = 16
NW = NUM_CORES * NUM_SUBCORES

N_ROWS = B * L
ROWS_PER_W = N_ROWS // NW
WIN = 40
NSEQ = 4
CHUNK = NSEQ * WIN
GROW = WIN
NG = CHUNK // GROW
NPHASE = L // WIN
CHUNKS_PER_W = ROWS_PER_W // CHUNK
NBUF = 4
NCOL = D_MODEL // 16


def _pos_encoding() -> np.ndarray:
    pos = np.arange(MAX_LEN, dtype=np.float64)[:, None]
    i = np.arange(0, D_MODEL, 2, dtype=np.float64)[None, :]
    loc = pos / (10000.0 ** (i / D_MODEL))
    enc = np.zeros((MAX_LEN, D_MODEL), dtype=np.float32)
    enc[:, 0::2] = np.sin(loc)
    enc[:, 1::2] = np.cos(loc)
    return enc[:L]


_ENC = _pos_encoding()


def _sc_kernel():
    mesh = plsc.VectorSubcoreMesh(core_axis_name="c", subcore_axis_name="s")

    @functools.partial(
        pl.kernel,
        mesh=mesh,
        out_type=jax.ShapeDtypeStruct((B, L, D_MODEL), jnp.float32),
        scratch_types=[
            pltpu.VMEM((NG * CHUNKS_PER_W, GROW), jnp.int32),
            pltpu.VMEM((L, D_MODEL), jnp.float32),
        ]
        + [pltpu.VMEM((CHUNK, D_MODEL), jnp.float32) for _ in range(NBUF)]
        + [pltpu.SemaphoreType.DMA for _ in range(2 * NBUF + 1)],
    )
    def k(table_hbm, xr_hbm, enc_hbm, out_hbm, idx_v, enc_v,
          b0, b1, b2, b3, g0, g1, g2, g3, s0, s1, s2, s3, esem):
        bufs = (b0, b1, b2, b3)
        gsems = (g0, g1, g2, g3)
        ssems = (s0, s1, s2, s3)
        wid = lax.axis_index("s") * NUM_CORES + lax.axis_index("c")
        irow0 = wid * NG * CHUNKS_PER_W
        seq0 = wid * (ROWS_PER_W // L)

        enc_copy = pltpu.make_async_copy(enc_hbm, enc_v, esem)
        enc_copy.start()
        pltpu.sync_copy(xr_hbm.at[pl.ds(irow0, NG * CHUNKS_PER_W)], idx_v)

        def idx_row(c, g):
            q = lax.div(c, NPHASE)
            f = lax.rem(c, NPHASE)
            return (NSEQ * q + g) * NPHASE + f

        def start_gather(c, b):
            for g in range(NG):
                pltpu.make_async_copy(
                    table_hbm.at[idx_v.at[idx_row(c, g)]],
                    bufs[b].at[pl.ds(g * GROW, GROW)], gsems[b]).start()

        def wait_gather(c, b):
            for g in range(NG):
                pltpu.make_async_copy(
                    table_hbm.at[idx_v.at[idx_row(c, g)]],
                    bufs[b].at[pl.ds(g * GROW, GROW)], gsems[b]).wait()

        def out_dst(c, half):
            q = lax.div(c, NPHASE)
            f = lax.rem(c, NPHASE)
            return out_hbm.at[seq0 + NSEQ * q + half, pl.ds(f * WIN, WIN)]

        def start_store(c, b):
            for half in range(NSEQ):
                pltpu.make_async_copy(bufs[b].at[pl.ds(half * WIN, WIN)],
                                      out_dst(c, half), ssems[b]).start()

        def wait_store(c, b):
            for half in range(NSEQ):
                pltpu.make_async_copy(bufs[b].at[pl.ds(half * WIN, WIN)],
                                      out_dst(c, half), ssems[b]).wait()

        def add_enc(c, b):
            base = lax.rem(c, NPHASE) * WIN

            def row_body(r, _):
                cols = range(NCOL)
                e = [enc_v[base + r, pl.ds(col * 16, 16)] for col in cols]
                for half in range(NSEQ):
                    row = half * WIN + r
                    g = [bufs[b][row, pl.ds(col * 16, 16)] for col in cols]
                    for col in cols:
                        bufs[b][row, pl.ds(col * 16, 16)] = g[col] + e[col]
                return 0

            lax.fori_loop(0, WIN, row_body, 0, unroll=2)

        for b in range(NBUF):
            start_gather(b, b)
        enc_copy.wait()

        def outer(i, _):
            c0 = i * NBUF
            for b in range(NBUF):
                c = c0 + b
                wait_gather(c, b)
                add_enc(c, b)
                start_store(c, b)
            for b in range(NBUF):
                c = c0 + b

                @pl.when(c + NBUF < CHUNKS_PER_W)
                def _():
                    wait_store(c, b)
                    start_gather(c + NBUF, b)
            return 0

        lax.fori_loop(0, CHUNKS_PER_W // NBUF, outer, 0)

        for b in range(NBUF):
            c = CHUNKS_PER_W - NBUF + b
            wait_store(c, b)

    return k


_K = _sc_kernel()


def kernel(x, embed_table):
    xr = jnp.asarray(x, jnp.int32).reshape(N_ROWS // GROW, GROW)
    enc = jnp.asarray(_ENC)
    return _K(embed_table, xr, enc)


# speedup vs baseline: 1.0275x; 1.0275x over previous
import functools

import jax
import jax.numpy as jnp
import numpy as np
from jax import lax
from jax.experimental import pallas as pl
from jax.experimental.pallas import tpu as pltpu
from jax.experimental.pallas import tpu_sc as plsc

VOCAB = 100000
D_MODEL = 128
MAX_LEN = 512
B = 1024
L = 200

NUM_CORES = 2
NUM_SUBCORES = 16
NW = NUM_CORES * NUM_SUBCORES

N_ROWS = B * L
ROWS_PER_W = N_ROWS // NW
WIN = 40
NSEQ = 4
CHUNK = NSEQ * WIN
GROW = WIN
NG = CHUNK // GROW
NPHASE = L // WIN
CHUNKS_PER_W = ROWS_PER_W // CHUNK
NBUF = 4
NCOL = D_MODEL // 16


def _pos_encoding() -> np.ndarray:
    pos = np.arange(MAX_LEN, dtype=np.float64)[:, None]
    i = np.arange(0, D_MODEL, 2, dtype=np.float64)[None, :]
    loc = pos / (10000.0 ** (i / D_MODEL))
    enc = np.zeros((MAX_LEN, D_MODEL), dtype=np.float32)
    enc[:, 0::2] = np.sin(loc)
    enc[:, 1::2] = np.cos(loc)
    return enc[:L]


_ENC = _pos_encoding()


def _sc_kernel():
    mesh = plsc.VectorSubcoreMesh(core_axis_name="c", subcore_axis_name="s")

    @functools.partial(
        pl.kernel,
        mesh=mesh,
        out_type=jax.ShapeDtypeStruct((B, L, D_MODEL), jnp.float32),
        scratch_types=[
            pltpu.VMEM((NG * CHUNKS_PER_W, GROW), jnp.int32),
            pltpu.VMEM((L, D_MODEL), jnp.float32),
        ]
        + [pltpu.VMEM((CHUNK, D_MODEL), jnp.float32) for _ in range(NBUF)]
        + [pltpu.SemaphoreType.DMA for _ in range(2 * NBUF + 1)],
    )
    def k(table_hbm, xr_hbm, enc_hbm, out_hbm, idx_v, enc_v,
          b0, b1, b2, b3, g0, g1, g2, g3, s0, s1, s2, s3, esem):
        bufs = (b0, b1, b2, b3)
        gsems = (g0, g1, g2, g3)
        ssems = (s0, s1, s2, s3)
        wid = lax.axis_index("s") * NUM_CORES + lax.axis_index("c")
        irow0 = wid * NG * CHUNKS_PER_W
        seq0 = wid * (ROWS_PER_W // L)

        enc_copy = pltpu.make_async_copy(enc_hbm, enc_v, esem)
        enc_copy.start()
        pltpu.sync_copy(xr_hbm.at[pl.ds(irow0, NG * CHUNKS_PER_W)], idx_v)

        def idx_row(c, g):
            q = lax.div(c, NPHASE)
            f = lax.rem(c, NPHASE)
            return (NSEQ * q + g) * NPHASE + f

        def start_gather(c, b):
            for g in range(NG):
                pltpu.make_async_copy(
                    table_hbm.at[idx_v.at[idx_row(c, g)]],
                    bufs[b].at[pl.ds(g * GROW, GROW)], gsems[b]).start()

        def wait_gather(c, b):
            for g in range(NG):
                pltpu.make_async_copy(
                    table_hbm.at[idx_v.at[idx_row(c, g)]],
                    bufs[b].at[pl.ds(g * GROW, GROW)], gsems[b]).wait()

        def out_dst(c, half):
            q = lax.div(c, NPHASE)
            f = lax.rem(c, NPHASE)
            return out_hbm.at[seq0 + NSEQ * q + half, pl.ds(f * WIN, WIN)]

        def start_store(c, b):
            for half in range(NSEQ):
                pltpu.make_async_copy(bufs[b].at[pl.ds(half * WIN, WIN)],
                                      out_dst(c, half), ssems[b]).start()

        def wait_store(c, b):
            for half in range(NSEQ):
                pltpu.make_async_copy(bufs[b].at[pl.ds(half * WIN, WIN)],
                                      out_dst(c, half), ssems[b]).wait()

        def add_enc(c, b):
            base = lax.rem(c, NPHASE) * WIN

            def row_body(r, _):
                cols = range(NCOL)
                e = [enc_v[base + r, pl.ds(col * 16, 16)] for col in cols]
                for half in range(NSEQ):
                    row = half * WIN + r
                    g = [bufs[b][row, pl.ds(col * 16, 16)] for col in cols]
                    for col in cols:
                        bufs[b][row, pl.ds(col * 16, 16)] = g[col] + e[col]
                return 0

            lax.fori_loop(0, WIN, row_body, 0)

        for b in range(NBUF):
            start_gather(b, b)
        enc_copy.wait()

        def outer(i, _):
            c0 = i * NBUF
            for b in range(NBUF):
                c = c0 + b
                wait_gather(c, b)
                add_enc(c, b)
                start_store(c, b)
            for b in range(NBUF):
                c = c0 + b

                @pl.when(c + NBUF < CHUNKS_PER_W)
                def _():
                    wait_store(c, b)
                    start_gather(c + NBUF, b)
            return 0

        lax.fori_loop(0, CHUNKS_PER_W // NBUF, outer, 0)

        for b in range(NBUF):
            c = CHUNKS_PER_W - NBUF + b
            wait_store(c, b)

    return k


_K = _sc_kernel()


def kernel(x, embed_table):
    xr = jnp.asarray(x, jnp.int32).reshape(N_ROWS // GROW, GROW)
    enc = jnp.asarray(_ENC)
    return _K(embed_table, xr, enc)
